# baseline (speedup 1.0000x reference)
import functools

import jax
import jax.numpy as jnp
from jax import lax
from jax.experimental import pallas as pl
from jax.experimental.pallas import tpu as pltpu
from jax.experimental.pallas import tpu_sc as plsc

N = 10000
E = 320000
D = 128

NC = 2
NS = 16
L = 16
NW = NC * NS
CB = 64
NCH0 = 181
NCH1 = 133
EPP = (NCH0 + NCH1) * CB
EPAD = NS * EPP - E
ZROWS = 200
ZB = 40
NOCH = N // ZROWS
NOCH_PER_TILE = -(-NOCH // NS)

BN = 2000


def _tc_prep_body(x_ref, w_ref, b_ref, at_ref, ab_ref, h_ref, st_ref):
    h = lax.dot_general(x_ref[...], w_ref[...], (((1,), (1,)), ((), ())),
                        preferred_element_type=jnp.float32)
    h = h + b_ref[...]
    h_ref[...] = h
    st = lax.dot_general(h, at_ref[...], (((1,), (1,)), ((), ())),
                         preferred_element_type=jnp.float32)
    st_ref[...] = st + ab_ref[...]


def _tc_combine_body(p0_ref, p1_ref, zp_ref, o_ref):
    z = jnp.sum(zp_ref[...])
    o_ref[...] = (p0_ref[0] + p1_ref[0]) * (1.0 / z)


def _sc_edges_body(h_hbm, sti_hbm, row_hbm, col_hbm,
                   part_hbm, zp_hbm,
                   acc_sh, sti_v,
                   row_v0, col_v0, row_v1, col_v1,
                   row_v2, col_v2, row_v3, col_v3,
                   rows0, rows1, scat0, scat1, zv,
                   si0, si1, si2, si3, sg0, sg1, ss0, ss1):
    cid = lax.axis_index("c")
    sid = lax.axis_index("s")
    wid = sid * NC + cid

    rowb = (row_v0, row_v1, row_v2, row_v3)
    colb = (col_v0, col_v1, col_v2, col_v3)
    rowsb = (rows0, rows1)
    scatb = (scat0, scat1)
    sis = (si0, si1, si2, si3)
    sgs = (sg0, sg1)
    sss = (ss0, ss1)

    pltpu.sync_copy(sti_hbm, sti_v)

    z16 = jnp.zeros((L,), jnp.float32)

    @pl.loop(0, ZB)
    def _zero_stage(i):
        for q in range(D // L):
            scat0[i, pl.ds(q * L, L)] = z16

    for j in range(NOCH_PER_TILE):
        ch = sid + NS * j

        @pl.when(ch < NOCH)
        def _():
            for q in range(ZROWS // ZB):
                pltpu.sync_copy(scat0.at[pl.ds(0, ZB)],
                                acc_sh.at[pl.ds(ch * ZROWS + q * ZB, ZB)])

    plsc.subcore_barrier()

    nch = jnp.where(cid == 0, NCH0, NCH1)
    ebase = sid * EPP + cid * (NCH0 * CB)
    himask = jnp.int32(-65536)

    def idx_start(g, p4):
        off = ebase + g * CB
        pltpu.async_copy(row_hbm.at[pl.ds(off, CB)], rowb[p4], sis[p4])
        pltpu.async_copy(col_hbm.at[pl.ds(off, CB)], colb[p4], sis[p4])

    def idx_wait(p4):
        pltpu.make_async_copy(row_hbm.at[pl.ds(0, CB)], rowb[p4], sis[p4]).wait()
        pltpu.make_async_copy(col_hbm.at[pl.ds(0, CB)], colb[p4], sis[p4]).wait()

    def gather_start(p4, p2):
        pltpu.async_copy(h_hbm.at[colb[p4]], rowsb[p2], sgs[p2])

    def gather_wait(p2):
        pltpu.make_async_copy(h_hbm.at[colb[0]], rowsb[p2], sgs[p2]).wait()

    def scat_start(p4, s):
        pltpu.async_copy(scatb[s], acc_sh.at[rowb[p4]], sss[s], add=True)

    def scat_wait(p4, s):
        pltpu.make_async_copy(scatb[s], acc_sh.at[rowb[p4]], sss[s]).wait()

    def compute(p2, p4, s, zacc):
        R = rowsb[p2]
        S = scatb[s]

        @pl.loop(0, CB // L, init_carry=zacc)
        def grp(i, zacc):
            rv = rowb[p4][pl.ds(i * L, L)]
            cv = colb[p4][pl.ds(i * L, L)]
            wr = plsc.load_gather(sti_v, [rv])
            wc = plsc.load_gather(sti_v, [cv])
            sv = plsc.bitcast(lax.shift_left(wr, 16), jnp.float32)
            tv = plsc.bitcast(jnp.bitwise_and(wc, himask), jnp.float32)
            e = sv + tv
            wv = jnp.exp(jnp.where(e > 0, e, 0.2 * e))
            zacc = zacc + wv
            for j in range(L):
                wj = wv.at[jnp.full((L,), j, jnp.int32)].get(
                    mode="promise_in_bounds")
                k = i * L + j
                for q in range(D // L):
                    S[k, pl.ds(q * L, L)] = R[k, pl.ds(q * L, L)] * wj
            return zacc

        return grp

    def step(g, p2, p4, s, zacc, first=False, do_next=True):
        if not first:
            scat_wait((p4 + 2) % 4, s)
        if do_next:
            idx_start(g + 2, (p4 + 2) % 4)
        gather_wait(p2)
        zacc = compute(p2, p4, s, zacc)
        scat_start(p4, s)
        if do_next:
            idx_wait((p4 + 2) % 4)
            gather_start((p4 + 2) % 4, p2)
        return zacc

    idx_start(0, 0)
    idx_start(1, 1)
    idx_wait(0)
    gather_start(0, 0)
    idx_wait(1)
    gather_start(1, 1)

    zacc = jnp.zeros((L,), jnp.float32)
    zacc = step(0, 0, 0, 0, zacc, first=True)
    zacc = step(1, 1, 1, 1, zacc, first=True)

    @pl.loop(0, (nch - 5) // 4, init_carry=zacc)
    def _main(m, zacc):
        g = 4 * m + 2
        zacc = step(g, 0, 2, 0, zacc)
        zacc = step(g + 1, 1, 3, 1, zacc)
        zacc = step(g + 2, 0, 0, 0, zacc)
        zacc = step(g + 3, 1, 1, 1, zacc)
        return zacc

    zacc = _main
    zacc = step(nch - 3, 0, 2, 0, zacc)
    zacc = step(nch - 2, 1, 3, 1, zacc, do_next=False)
    zacc = step(nch - 1, 0, 0, 0, zacc, do_next=False)
    scat_wait(3, 1)
    scat_wait(0, 0)

    zv[0, 0, :] = zacc
    pltpu.sync_copy(zv, zp_hbm.at[pl.ds(wid, 1)])

    plsc.subcore_barrier()
    for j in range(NOCH_PER_TILE):
        ch = sid + NS * j

        @pl.when(ch < NOCH)
        def _():
            sl = pl.ds(ch * ZROWS, ZROWS)
            pltpu.sync_copy(acc_sh.at[sl], part_hbm.at[cid, sl])


_sc_edges = functools.partial(
    pl.kernel,
    out_type=(
        jax.ShapeDtypeStruct((NC, N, D), jnp.float32),
        jax.ShapeDtypeStruct((NW, 1, L), jnp.float32),
    ),
    mesh=plsc.VectorSubcoreMesh(core_axis_name="c", subcore_axis_name="s"),
    compiler_params=pltpu.CompilerParams(needs_layout_passes=False),
    scratch_types=(
        pltpu.VMEM_SHARED((N + 8, D), jnp.float32),
        pltpu.VMEM((N + 8,), jnp.int32),
        pltpu.VMEM((CB,), jnp.int32),
        pltpu.VMEM((CB,), jnp.int32),
        pltpu.VMEM((CB,), jnp.int32),
        pltpu.VMEM((CB,), jnp.int32),
        pltpu.VMEM((CB,), jnp.int32),
        pltpu.VMEM((CB,), jnp.int32),
        pltpu.VMEM((CB,), jnp.int32),
        pltpu.VMEM((CB,), jnp.int32),
        pltpu.VMEM((CB, D), jnp.float32),
        pltpu.VMEM((CB, D), jnp.float32),
        pltpu.VMEM((CB, D), jnp.float32),
        pltpu.VMEM((CB, D), jnp.float32),
        pltpu.VMEM((1, 1, L), jnp.float32),
        pltpu.SemaphoreType.DMA,
        pltpu.SemaphoreType.DMA,
        pltpu.SemaphoreType.DMA,
        pltpu.SemaphoreType.DMA,
        pltpu.SemaphoreType.DMA,
        pltpu.SemaphoreType.DMA,
        pltpu.SemaphoreType.DMA,
        pltpu.SemaphoreType.DMA,
    ),
)(_sc_edges_body)


@jax.jit
def kernel(x, edge_index, W_weight, W_bias, a_weight, a_bias):
    row = edge_index[0].astype(jnp.int32)
    col = edge_index[1].astype(jnp.int32)
    at = a_weight.reshape(2, D).astype(jnp.float32)
    ab = jnp.stack([a_bias[0], jnp.zeros((), jnp.float32)]).reshape(1, 2)
    b = W_bias.reshape(1, D)

    h, st = pl.pallas_call(
        _tc_prep_body,
        grid=(N // BN,),
        in_specs=[
            pl.BlockSpec((BN, D), lambda i: (i, 0)),
            pl.BlockSpec((D, D), lambda i: (0, 0)),
            pl.BlockSpec((1, D), lambda i: (0, 0)),
            pl.BlockSpec((2, D), lambda i: (0, 0)),
            pl.BlockSpec((1, 2), lambda i: (0, 0)),
        ],
        out_specs=[
            pl.BlockSpec((BN, D), lambda i: (i, 0)),
            pl.BlockSpec((BN, 2), lambda i: (i, 0)),
        ],
        out_shape=[
            jax.ShapeDtypeStruct((N, D), jnp.float32),
            jax.ShapeDtypeStruct((N, 2), jnp.float32),
        ],
    )(x, W_weight, b, at, ab)

    st_pad = jnp.concatenate(
        [st, jnp.full((8, 2), -jnp.inf, jnp.float32)], axis=0)
    sti = lax.bitcast_convert_type(st_pad.astype(jnp.bfloat16), jnp.int32)
    row_pad = jnp.concatenate([row, jnp.full((EPAD,), N, jnp.int32)])
    col_pad = jnp.concatenate([col, jnp.zeros((EPAD,), jnp.int32)])
    part, zp = _sc_edges(h, sti, row_pad, col_pad)

    out = pl.pallas_call(
        _tc_combine_body,
        grid=(N // BN,),
        in_specs=[
            pl.BlockSpec((1, BN, D), lambda i: (0, i, 0)),
            pl.BlockSpec((1, BN, D), lambda i: (1, i, 0)),
            pl.BlockSpec((4, 128), lambda i: (0, 0)),
        ],
        out_specs=pl.BlockSpec((BN, D), lambda i: (i, 0)),
        out_shape=jax.ShapeDtypeStruct((N, D), jnp.float32),
    )(part, part, zp.reshape(4, 128))
    return out


# speedup vs baseline: 1.0041x; 1.0041x over previous
import functools

import jax
import jax.numpy as jnp
from jax import lax
from jax.experimental import pallas as pl
from jax.experimental.pallas import tpu as pltpu
from jax.experimental.pallas import tpu_sc as plsc

N = 10000
E = 320000
D = 128

NC = 2
NS = 16
L = 16
NW = NC * NS
CB = 64
NCH0 = 185
NCH1 = 129
EPP = (NCH0 + NCH1) * CB
EPAD = NS * EPP - E
ZROWS = 200
ZB = 40
NOCH = N // ZROWS
NOCH_PER_TILE = -(-NOCH // NS)

BN = 2000


def _tc_prep_body(x_ref, w_ref, b_ref, at_ref, ab_ref, h_ref, st_ref):
    h = lax.dot_general(x_ref[...], w_ref[...], (((1,), (1,)), ((), ())),
                        preferred_element_type=jnp.float32)
    h = h + b_ref[...]
    h_ref[...] = h
    st = lax.dot_general(h, at_ref[...], (((1,), (1,)), ((), ())),
                         preferred_element_type=jnp.float32)
    st_ref[...] = st + ab_ref[...]


def _tc_combine_body(p0_ref, p1_ref, zp_ref, o_ref):
    z = jnp.sum(zp_ref[...])
    o_ref[...] = (p0_ref[0] + p1_ref[0]) * (1.0 / z)


def _sc_edges_body(h_hbm, sti_hbm, row_hbm, col_hbm,
                   part_hbm, zp_hbm,
                   acc_sh, sti_v,
                   row_v0, col_v0, row_v1, col_v1,
                   row_v2, col_v2, row_v3, col_v3,
                   rows0, rows1, scat0, scat1, zv,
                   si0, si1, si2, si3, sg0, sg1, ss0, ss1):
    cid = lax.axis_index("c")
    sid = lax.axis_index("s")
    wid = sid * NC + cid

    rowb = (row_v0, row_v1, row_v2, row_v3)
    colb = (col_v0, col_v1, col_v2, col_v3)
    rowsb = (rows0, rows1)
    scatb = (scat0, scat1)
    sis = (si0, si1, si2, si3)
    sgs = (sg0, sg1)
    sss = (ss0, ss1)

    pltpu.sync_copy(sti_hbm, sti_v)

    z16 = jnp.zeros((L,), jnp.float32)

    @pl.loop(0, ZB)
    def _zero_stage(i):
        for q in range(D // L):
            scat0[i, pl.ds(q * L, L)] = z16

    for j in range(NOCH_PER_TILE):
        ch = sid + NS * j

        @pl.when(ch < NOCH)
        def _():
            for q in range(ZROWS // ZB):
                pltpu.sync_copy(scat0.at[pl.ds(0, ZB)],
                                acc_sh.at[pl.ds(ch * ZROWS + q * ZB, ZB)])

    plsc.subcore_barrier()

    nch = jnp.where(cid == 0, NCH0, NCH1)
    ebase = sid * EPP + cid * (NCH0 * CB)
    himask = jnp.int32(-65536)

    def idx_start(g, p4):
        off = ebase + g * CB
        pltpu.async_copy(row_hbm.at[pl.ds(off, CB)], rowb[p4], sis[p4])
        pltpu.async_copy(col_hbm.at[pl.ds(off, CB)], colb[p4], sis[p4])

    def idx_wait(p4):
        pltpu.make_async_copy(row_hbm.at[pl.ds(0, CB)], rowb[p4], sis[p4]).wait()
        pltpu.make_async_copy(col_hbm.at[pl.ds(0, CB)], colb[p4], sis[p4]).wait()

    def gather_start(p4, p2):
        pltpu.async_copy(h_hbm.at[colb[p4]], rowsb[p2], sgs[p2])

    def gather_wait(p2):
        pltpu.make_async_copy(h_hbm.at[colb[0]], rowsb[p2], sgs[p2]).wait()

    def scat_start(p4, s):
        pltpu.async_copy(scatb[s], acc_sh.at[rowb[p4]], sss[s], add=True)

    def scat_wait(p4, s):
        pltpu.make_async_copy(scatb[s], acc_sh.at[rowb[p4]], sss[s]).wait()

    def compute(p2, p4, s, zacc):
        R = rowsb[p2]
        S = scatb[s]

        @pl.loop(0, CB // L, init_carry=zacc)
        def grp(i, zacc):
            rv = rowb[p4][pl.ds(i * L, L)]
            cv = colb[p4][pl.ds(i * L, L)]
            wr = plsc.load_gather(sti_v, [rv])
            wc = plsc.load_gather(sti_v, [cv])
            sv = plsc.bitcast(lax.shift_left(wr, 16), jnp.float32)
            tv = plsc.bitcast(jnp.bitwise_and(wc, himask), jnp.float32)
            e = sv + tv
            wv = jnp.exp(jnp.where(e > 0, e, 0.2 * e))
            zacc = zacc + wv
            for j in range(L):
                wj = wv.at[jnp.full((L,), j, jnp.int32)].get(
                    mode="promise_in_bounds")
                k = i * L + j
                for q in range(D // L):
                    S[k, pl.ds(q * L, L)] = R[k, pl.ds(q * L, L)] * wj
            return zacc

        return grp

    def step(g, p2, p4, s, zacc, first=False, do_next=True):
        if not first:
            scat_wait((p4 + 2) % 4, s)
        if do_next:
            idx_start(g + 2, (p4 + 2) % 4)
        gather_wait(p2)
        zacc = compute(p2, p4, s, zacc)
        scat_start(p4, s)
        if do_next:
            idx_wait((p4 + 2) % 4)
            gather_start((p4 + 2) % 4, p2)
        return zacc

    idx_start(0, 0)
    idx_start(1, 1)
    idx_wait(0)
    gather_start(0, 0)
    idx_wait(1)
    gather_start(1, 1)

    zacc = jnp.zeros((L,), jnp.float32)
    zacc = step(0, 0, 0, 0, zacc, first=True)
    zacc = step(1, 1, 1, 1, zacc, first=True)

    @pl.loop(0, (nch - 5) // 4, init_carry=zacc)
    def _main(m, zacc):
        g = 4 * m + 2
        zacc = step(g, 0, 2, 0, zacc)
        zacc = step(g + 1, 1, 3, 1, zacc)
        zacc = step(g + 2, 0, 0, 0, zacc)
        zacc = step(g + 3, 1, 1, 1, zacc)
        return zacc

    zacc = _main
    zacc = step(nch - 3, 0, 2, 0, zacc)
    zacc = step(nch - 2, 1, 3, 1, zacc, do_next=False)
    zacc = step(nch - 1, 0, 0, 0, zacc, do_next=False)
    scat_wait(3, 1)
    scat_wait(0, 0)

    zv[0, 0, :] = zacc
    pltpu.sync_copy(zv, zp_hbm.at[pl.ds(wid, 1)])

    plsc.subcore_barrier()
    for j in range(NOCH_PER_TILE):
        ch = sid + NS * j

        @pl.when(ch < NOCH)
        def _():
            sl = pl.ds(ch * ZROWS, ZROWS)
            pltpu.sync_copy(acc_sh.at[sl], part_hbm.at[cid, sl])


_sc_edges = functools.partial(
    pl.kernel,
    out_type=(
        jax.ShapeDtypeStruct((NC, N, D), jnp.float32),
        jax.ShapeDtypeStruct((NW, 1, L), jnp.float32),
    ),
    mesh=plsc.VectorSubcoreMesh(core_axis_name="c", subcore_axis_name="s"),
    compiler_params=pltpu.CompilerParams(needs_layout_passes=False),
    scratch_types=(
        pltpu.VMEM_SHARED((N + 8, D), jnp.float32),
        pltpu.VMEM((N + 8,), jnp.int32),
        pltpu.VMEM((CB,), jnp.int32),
        pltpu.VMEM((CB,), jnp.int32),
        pltpu.VMEM((CB,), jnp.int32),
        pltpu.VMEM((CB,), jnp.int32),
        pltpu.VMEM((CB,), jnp.int32),
        pltpu.VMEM((CB,), jnp.int32),
        pltpu.VMEM((CB,), jnp.int32),
        pltpu.VMEM((CB,), jnp.int32),
        pltpu.VMEM((CB, D), jnp.float32),
        pltpu.VMEM((CB, D), jnp.float32),
        pltpu.VMEM((CB, D), jnp.float32),
        pltpu.VMEM((CB, D), jnp.float32),
        pltpu.VMEM((1, 1, L), jnp.float32),
        pltpu.SemaphoreType.DMA,
        pltpu.SemaphoreType.DMA,
        pltpu.SemaphoreType.DMA,
        pltpu.SemaphoreType.DMA,
        pltpu.SemaphoreType.DMA,
        pltpu.SemaphoreType.DMA,
        pltpu.SemaphoreType.DMA,
        pltpu.SemaphoreType.DMA,
    ),
)(_sc_edges_body)


@jax.jit
def kernel(x, edge_index, W_weight, W_bias, a_weight, a_bias):
    row = edge_index[0].astype(jnp.int32)
    col = edge_index[1].astype(jnp.int32)
    at = a_weight.reshape(2, D).astype(jnp.float32)
    ab = jnp.stack([a_bias[0], jnp.zeros((), jnp.float32)]).reshape(1, 2)
    b = W_bias.reshape(1, D)

    h, st = pl.pallas_call(
        _tc_prep_body,
        grid=(1,),
        in_specs=[
            pl.BlockSpec((N, D), lambda i: (0, 0)),
            pl.BlockSpec((D, D), lambda i: (0, 0)),
            pl.BlockSpec((1, D), lambda i: (0, 0)),
            pl.BlockSpec((2, D), lambda i: (0, 0)),
            pl.BlockSpec((1, 2), lambda i: (0, 0)),
        ],
        out_specs=[
            pl.BlockSpec((N, D), lambda i: (0, 0)),
            pl.BlockSpec((N, 2), lambda i: (0, 0)),
        ],
        out_shape=[
            jax.ShapeDtypeStruct((N, D), jnp.float32),
            jax.ShapeDtypeStruct((N, 2), jnp.float32),
        ],
    )(x, W_weight, b, at, ab)

    st_pad = jnp.concatenate(
        [st, jnp.full((8, 2), -jnp.inf, jnp.float32)], axis=0)
    sti = lax.bitcast_convert_type(st_pad.astype(jnp.bfloat16), jnp.int32)
    row_pad = jnp.concatenate([row, jnp.full((EPAD,), N, jnp.int32)])
    col_pad = jnp.concatenate([col, jnp.zeros((EPAD,), jnp.int32)])
    part, zp = _sc_edges(h, sti, row_pad, col_pad)

    out = pl.pallas_call(
        _tc_combine_body,
        grid=(N // BN,),
        in_specs=[
            pl.BlockSpec((1, BN, D), lambda i: (0, i, 0)),
            pl.BlockSpec((1, BN, D), lambda i: (1, i, 0)),
            pl.BlockSpec((4, 128), lambda i: (0, 0)),
        ],
        out_specs=pl.BlockSpec((BN, D), lambda i: (i, 0)),
        out_shape=jax.ShapeDtypeStruct((N, D), jnp.float32),
    )(part, part, zp.reshape(4, 128))
    return out


# speedup vs baseline: 1.1440x; 1.1393x over previous
import functools

import jax
import jax.numpy as jnp
from jax import lax
from jax.experimental import pallas as pl
from jax.experimental.pallas import tpu as pltpu
from jax.experimental.pallas import tpu_sc as plsc

N = 10000
E = 320000
D = 128

NC = 2
NS = 16
L = 16
NW = NC * NS
CB = 64
NCH0 = 185
NCH1 = 129
EPP = (NCH0 + NCH1) * CB
EPAD = NS * EPP - E
ZROWS = 200
ZB = 40
NOCH = N // ZROWS
NOCH_PER_TILE = -(-NOCH // NS)

BN = 2000


def _tc_prep_body(x_ref, w_ref, b_ref, at_ref, ab_ref, h_ref, st_ref):
    h = lax.dot_general(x_ref[...], w_ref[...], (((1,), (1,)), ((), ())),
                        preferred_element_type=jnp.float32)
    h = h + b_ref[...]
    h_ref[...] = h
    st = lax.dot_general(h, at_ref[...], (((1,), (1,)), ((), ())),
                         preferred_element_type=jnp.float32)
    st_ref[...] = st + ab_ref[...]


def _tc_combine_body(p0_ref, p1_ref, zp_ref, o_ref):
    z = jnp.sum(zp_ref[...])
    o_ref[...] = (p0_ref[0] + p1_ref[0]) * (1.0 / z)


def _sc_edges_body(h_hbm, sti_hbm, row_hbm, col_hbm,
                   part_hbm, zp_hbm,
                   acc_sh, sti_v,
                   row_v0, col_v0, row_v1, col_v1,
                   row_v2, col_v2, row_v3, col_v3,
                   rows0, rows1, scat0, scat1, zv,
                   si0, si1, si2, si3, sg0, sg1, ss0, ss1):
    cid = lax.axis_index("c")
    sid = lax.axis_index("s")
    wid = sid * NC + cid

    rowb = (row_v0, row_v1, row_v2, row_v3)
    colb = (col_v0, col_v1, col_v2, col_v3)
    rowsb = (rows0, rows1)
    scatb = (scat0, scat1)
    sis = (si0, si1, si2, si3)
    sgs = (sg0, sg1)
    sss = (ss0, ss1)

    pltpu.sync_copy(sti_hbm, sti_v)

    z16 = jnp.zeros((L,), jnp.float32)

    @pl.loop(0, ZB)
    def _zero_stage(i):
        for q in range(D // L):
            scat0[i, pl.ds(q * L, L)] = z16

    for j in range(NOCH_PER_TILE):
        ch = sid + NS * j

        @pl.when(ch < NOCH)
        def _():
            for q in range(ZROWS // ZB):
                pltpu.sync_copy(scat0.at[pl.ds(0, ZB)],
                                acc_sh.at[pl.ds(ch * ZROWS + q * ZB, ZB)])

    plsc.subcore_barrier()

    nch = jnp.where(cid == 0, NCH0, NCH1)
    ebase = sid * EPP + cid * (NCH0 * CB)
    himask = jnp.int32(-65536)

    def idx_start(g, p4):
        off = jnp.minimum(ebase + g * CB, E - CB)
        pltpu.async_copy(row_hbm.at[pl.ds(off, CB)], rowb[p4], sis[p4])
        pltpu.async_copy(col_hbm.at[pl.ds(off, CB)], colb[p4], sis[p4])

    def idx_wait(p4):
        pltpu.make_async_copy(row_hbm.at[pl.ds(0, CB)], rowb[p4], sis[p4]).wait()
        pltpu.make_async_copy(col_hbm.at[pl.ds(0, CB)], colb[p4], sis[p4]).wait()

    def gather_start(p4, p2):
        pltpu.async_copy(h_hbm.at[colb[p4]], rowsb[p2], sgs[p2])

    def gather_wait(p2):
        pltpu.make_async_copy(h_hbm.at[colb[0]], rowsb[p2], sgs[p2]).wait()

    def scat_start(p4, s):
        pltpu.async_copy(scatb[s], acc_sh.at[rowb[p4]], sss[s], add=True)

    def scat_wait(p4, s):
        pltpu.make_async_copy(scatb[s], acc_sh.at[rowb[p4]], sss[s]).wait()

    def compute(g, p2, p4, s, zacc):
        R = rowsb[p2]
        S = scatb[s]
        valid = jnp.where(ebase + g * CB < E, 1.0, 0.0).astype(jnp.float32)

        @pl.loop(0, CB // L, init_carry=zacc)
        def grp(i, zacc):
            rv = rowb[p4][pl.ds(i * L, L)]
            cv = colb[p4][pl.ds(i * L, L)]
            wr = plsc.load_gather(sti_v, [rv])
            wc = plsc.load_gather(sti_v, [cv])
            sv = plsc.bitcast(lax.shift_left(wr, 16), jnp.float32)
            tv = plsc.bitcast(jnp.bitwise_and(wc, himask), jnp.float32)
            e = sv + tv
            wv = jnp.exp(jnp.where(e > 0, e, 0.2 * e)) * valid
            zacc = zacc + wv
            for j in range(L):
                wj = wv.at[jnp.full((L,), j, jnp.int32)].get(
                    mode="promise_in_bounds")
                k = i * L + j
                for q in range(D // L):
                    S[k, pl.ds(q * L, L)] = R[k, pl.ds(q * L, L)] * wj
            return zacc

        return grp

    def step(g, p2, p4, s, zacc, first=False, do_next=True):
        if not first:
            scat_wait((p4 + 2) % 4, s)
        if do_next:
            idx_start(g + 2, (p4 + 2) % 4)
        gather_wait(p2)
        zacc = compute(g, p2, p4, s, zacc)
        scat_start(p4, s)
        if do_next:
            idx_wait((p4 + 2) % 4)
            gather_start((p4 + 2) % 4, p2)
        return zacc

    idx_start(0, 0)
    idx_start(1, 1)
    idx_wait(0)
    gather_start(0, 0)
    idx_wait(1)
    gather_start(1, 1)

    zacc = jnp.zeros((L,), jnp.float32)
    zacc = step(0, 0, 0, 0, zacc, first=True)
    zacc = step(1, 1, 1, 1, zacc, first=True)

    @pl.loop(0, (nch - 5) // 4, init_carry=zacc)
    def _main(m, zacc):
        g = 4 * m + 2
        zacc = step(g, 0, 2, 0, zacc)
        zacc = step(g + 1, 1, 3, 1, zacc)
        zacc = step(g + 2, 0, 0, 0, zacc)
        zacc = step(g + 3, 1, 1, 1, zacc)
        return zacc

    zacc = _main
    zacc = step(nch - 3, 0, 2, 0, zacc)
    zacc = step(nch - 2, 1, 3, 1, zacc, do_next=False)
    zacc = step(nch - 1, 0, 0, 0, zacc, do_next=False)
    scat_wait(3, 1)
    scat_wait(0, 0)

    zv[0, 0, :] = zacc
    pltpu.sync_copy(zv, zp_hbm.at[pl.ds(wid, 1)])

    plsc.subcore_barrier()
    for j in range(NOCH_PER_TILE):
        ch = sid + NS * j

        @pl.when(ch < NOCH)
        def _():
            sl = pl.ds(ch * ZROWS, ZROWS)
            pltpu.sync_copy(acc_sh.at[sl], part_hbm.at[cid, sl])


_sc_edges = functools.partial(
    pl.kernel,
    out_type=(
        jax.ShapeDtypeStruct((NC, N, D), jnp.float32),
        jax.ShapeDtypeStruct((NW, 1, L), jnp.float32),
    ),
    mesh=plsc.VectorSubcoreMesh(core_axis_name="c", subcore_axis_name="s"),
    compiler_params=pltpu.CompilerParams(needs_layout_passes=False),
    scratch_types=(
        pltpu.VMEM_SHARED((N, D), jnp.float32),
        pltpu.VMEM((N,), jnp.int32),
        pltpu.VMEM((CB,), jnp.int32),
        pltpu.VMEM((CB,), jnp.int32),
        pltpu.VMEM((CB,), jnp.int32),
        pltpu.VMEM((CB,), jnp.int32),
        pltpu.VMEM((CB,), jnp.int32),
        pltpu.VMEM((CB,), jnp.int32),
        pltpu.VMEM((CB,), jnp.int32),
        pltpu.VMEM((CB,), jnp.int32),
        pltpu.VMEM((CB, D), jnp.float32),
        pltpu.VMEM((CB, D), jnp.float32),
        pltpu.VMEM((CB, D), jnp.float32),
        pltpu.VMEM((CB, D), jnp.float32),
        pltpu.VMEM((1, 1, L), jnp.float32),
        pltpu.SemaphoreType.DMA,
        pltpu.SemaphoreType.DMA,
        pltpu.SemaphoreType.DMA,
        pltpu.SemaphoreType.DMA,
        pltpu.SemaphoreType.DMA,
        pltpu.SemaphoreType.DMA,
        pltpu.SemaphoreType.DMA,
        pltpu.SemaphoreType.DMA,
    ),
)(_sc_edges_body)


@jax.jit
def kernel(x, edge_index, W_weight, W_bias, a_weight, a_bias):
    row = edge_index[0].astype(jnp.int32)
    col = edge_index[1].astype(jnp.int32)
    at = a_weight.reshape(2, D).astype(jnp.float32)
    ab = jnp.stack([a_bias[0], jnp.zeros((), jnp.float32)]).reshape(1, 2)
    b = W_bias.reshape(1, D)

    h, st = pl.pallas_call(
        _tc_prep_body,
        grid=(N // BN,),
        in_specs=[
            pl.BlockSpec((BN, D), lambda i: (i, 0)),
            pl.BlockSpec((D, D), lambda i: (0, 0)),
            pl.BlockSpec((1, D), lambda i: (0, 0)),
            pl.BlockSpec((2, D), lambda i: (0, 0)),
            pl.BlockSpec((1, 2), lambda i: (0, 0)),
        ],
        out_specs=[
            pl.BlockSpec((BN, D), lambda i: (i, 0)),
            pl.BlockSpec((BN, 2), lambda i: (i, 0)),
        ],
        out_shape=[
            jax.ShapeDtypeStruct((N, D), jnp.float32),
            jax.ShapeDtypeStruct((N, 2), jnp.float32),
        ],
    )(x, W_weight, b, at, ab)

    sti = lax.bitcast_convert_type(st.astype(jnp.bfloat16), jnp.int32)
    part, zp = _sc_edges(h, sti, row, col)

    out = pl.pallas_call(
        _tc_combine_body,
        grid=(N // BN,),
        in_specs=[
            pl.BlockSpec((1, BN, D), lambda i: (0, i, 0)),
            pl.BlockSpec((1, BN, D), lambda i: (1, i, 0)),
            pl.BlockSpec((4, 128), lambda i: (0, 0)),
        ],
        out_specs=pl.BlockSpec((BN, D), lambda i: (i, 0)),
        out_shape=jax.ShapeDtypeStruct((N, D), jnp.float32),
    )(part, part, zp.reshape(4, 128))
    return out


# speedup vs baseline: 1.2870x; 1.1250x over previous
import functools

import jax
import jax.numpy as jnp
from jax import lax
from jax.experimental import pallas as pl
from jax.experimental.pallas import tpu as pltpu
from jax.experimental.pallas import tpu_sc as plsc

N = 10000
E = 320000
D = 128

NC = 2
NS = 16
L = 16
NW = NC * NS
CB = 64
NCH0 = 165
NCH1 = 149
EPP = (NCH0 + NCH1) * CB
EPAD = NS * EPP - E
ZROWS = 200
ZB = 40
NOCH = N // ZROWS
NOCH_PER_TILE = -(-NOCH // NS)

BN = 2000


def _tc_prep_body(x_ref, w_ref, b_ref, at_ref, ab_ref, h_ref, st_ref):
    h = lax.dot_general(x_ref[...], w_ref[...], (((1,), (1,)), ((), ())),
                        preferred_element_type=jnp.float32)
    h = h + b_ref[...]
    h_ref[...] = h
    st = lax.dot_general(h, at_ref[...], (((1,), (1,)), ((), ())),
                         preferred_element_type=jnp.float32)
    st_ref[...] = st + ab_ref[...]


def _tc_combine_body(p0_ref, p1_ref, zp_ref, o_ref):
    z = jnp.sum(zp_ref[...])
    o_ref[...] = (p0_ref[0] + p1_ref[0]) * (1.0 / z)


def _sc_edges_body(h_hbm, sti_hbm, e_hbm,
                   part_hbm, zp_hbm,
                   acc_sh, sti_v,
                   row_v0, col_v0, row_v1, col_v1,
                   row_v2, col_v2, row_v3, col_v3,
                   rows0, rows1, scat0, scat1, zv,
                   si0, si1, si2, si3, sg0, sg1, ss0, ss1):
    cid = lax.axis_index("c")
    sid = lax.axis_index("s")
    wid = sid * NC + cid

    rowb = (row_v0, row_v1, row_v2, row_v3)
    colb = (col_v0, col_v1, col_v2, col_v3)
    rowsb = (rows0, rows1)
    scatb = (scat0, scat1)
    sis = (si0, si1, si2, si3)
    sgs = (sg0, sg1)
    sss = (ss0, ss1)

    pltpu.sync_copy(sti_hbm, sti_v)

    z16 = jnp.zeros((L,), jnp.float32)

    @pl.loop(0, ZB)
    def _zero_stage(i):
        for q in range(D // L):
            scat0[i, pl.ds(q * L, L)] = z16

    for j in range(NOCH_PER_TILE):
        ch = sid + NS * j

        @pl.when(ch < NOCH)
        def _():
            for q in range(ZROWS // ZB):
                pltpu.sync_copy(scat0.at[pl.ds(0, ZB)],
                                acc_sh.at[pl.ds(ch * ZROWS + q * ZB, ZB)])

    plsc.subcore_barrier()

    nch = jnp.where(cid == 0, NCH0, NCH1)
    ebase = sid * EPP + cid * (NCH0 * CB)
    himask = jnp.int32(-65536)

    def idx_start(g, p4):
        off = jnp.minimum(ebase + g * CB, E - CB)
        pltpu.async_copy(e_hbm.at[pl.ds(off, CB)], rowb[p4], sis[p4])
        pltpu.async_copy(e_hbm.at[pl.ds(E + off, CB)], colb[p4], sis[p4])

    def idx_wait(p4):
        pltpu.make_async_copy(e_hbm.at[pl.ds(0, CB)], rowb[p4], sis[p4]).wait()
        pltpu.make_async_copy(e_hbm.at[pl.ds(0, CB)], colb[p4], sis[p4]).wait()

    def gather_start(p4, p2):
        pltpu.async_copy(h_hbm.at[colb[p4]], rowsb[p2], sgs[p2])

    def gather_wait(p2):
        pltpu.make_async_copy(h_hbm.at[colb[0]], rowsb[p2], sgs[p2]).wait()

    def scat_start(p4, s):
        pltpu.async_copy(scatb[s], acc_sh.at[rowb[p4]], sss[s], add=True)

    def scat_wait(p4, s):
        pltpu.make_async_copy(scatb[s], acc_sh.at[rowb[p4]], sss[s]).wait()

    def compute(g, p2, p4, s, zacc):
        R = rowsb[p2]
        S = scatb[s]
        valid = jnp.where(ebase + g * CB < E, 1.0, 0.0).astype(jnp.float32)

        @pl.loop(0, CB // L, init_carry=zacc)
        def grp(i, zacc):
            rv = rowb[p4][pl.ds(i * L, L)]
            cv = colb[p4][pl.ds(i * L, L)]
            wr = plsc.load_gather(sti_v, [rv])
            wc = plsc.load_gather(sti_v, [cv])
            sv = plsc.bitcast(lax.shift_left(wr, 16), jnp.float32)
            tv = plsc.bitcast(jnp.bitwise_and(wc, himask), jnp.float32)
            e = sv + tv
            wv = jnp.exp(jnp.where(e > 0, e, 0.2 * e)) * valid
            zacc = zacc + wv
            for j in range(L):
                wj = wv.at[jnp.full((L,), j, jnp.int32)].get(
                    mode="promise_in_bounds")
                k = i * L + j
                for q in range(D // L):
                    S[k, pl.ds(q * L, L)] = R[k, pl.ds(q * L, L)] * wj
            return zacc

        return grp

    def step(g, p2, p4, s, zacc, first=False, do_next=True):
        if not first:
            scat_wait((p4 + 2) % 4, s)
        if do_next:
            idx_start(g + 2, (p4 + 2) % 4)
        gather_wait(p2)
        zacc = compute(g, p2, p4, s, zacc)
        scat_start(p4, s)
        if do_next:
            idx_wait((p4 + 2) % 4)
            gather_start((p4 + 2) % 4, p2)
        return zacc

    idx_start(0, 0)
    idx_start(1, 1)
    idx_wait(0)
    gather_start(0, 0)
    idx_wait(1)
    gather_start(1, 1)

    zacc = jnp.zeros((L,), jnp.float32)
    zacc = step(0, 0, 0, 0, zacc, first=True)
    zacc = step(1, 1, 1, 1, zacc, first=True)

    @pl.loop(0, (nch - 5) // 4, init_carry=zacc)
    def _main(m, zacc):
        g = 4 * m + 2
        zacc = step(g, 0, 2, 0, zacc)
        zacc = step(g + 1, 1, 3, 1, zacc)
        zacc = step(g + 2, 0, 0, 0, zacc)
        zacc = step(g + 3, 1, 1, 1, zacc)
        return zacc

    zacc = _main
    zacc = step(nch - 3, 0, 2, 0, zacc)
    zacc = step(nch - 2, 1, 3, 1, zacc, do_next=False)
    zacc = step(nch - 1, 0, 0, 0, zacc, do_next=False)
    scat_wait(3, 1)
    scat_wait(0, 0)

    zv[0, 0, :] = zacc
    pltpu.sync_copy(zv, zp_hbm.at[pl.ds(wid, 1)])

    plsc.subcore_barrier()
    for j in range(NOCH_PER_TILE):
        ch = sid + NS * j

        @pl.when(ch < NOCH)
        def _():
            sl = pl.ds(ch * ZROWS, ZROWS)
            pltpu.sync_copy(acc_sh.at[sl], part_hbm.at[cid, sl])


_sc_edges = functools.partial(
    pl.kernel,
    out_type=(
        jax.ShapeDtypeStruct((NC, N, D), jnp.float32),
        jax.ShapeDtypeStruct((NW, 1, L), jnp.float32),
    ),
    mesh=plsc.VectorSubcoreMesh(core_axis_name="c", subcore_axis_name="s"),
    compiler_params=pltpu.CompilerParams(needs_layout_passes=False),
    scratch_types=(
        pltpu.VMEM_SHARED((N, D), jnp.float32),
        pltpu.VMEM((N,), jnp.int32),
        pltpu.VMEM((CB,), jnp.int32),
        pltpu.VMEM((CB,), jnp.int32),
        pltpu.VMEM((CB,), jnp.int32),
        pltpu.VMEM((CB,), jnp.int32),
        pltpu.VMEM((CB,), jnp.int32),
        pltpu.VMEM((CB,), jnp.int32),
        pltpu.VMEM((CB,), jnp.int32),
        pltpu.VMEM((CB,), jnp.int32),
        pltpu.VMEM((CB, D), jnp.float32),
        pltpu.VMEM((CB, D), jnp.float32),
        pltpu.VMEM((CB, D), jnp.float32),
        pltpu.VMEM((CB, D), jnp.float32),
        pltpu.VMEM((1, 1, L), jnp.float32),
        pltpu.SemaphoreType.DMA,
        pltpu.SemaphoreType.DMA,
        pltpu.SemaphoreType.DMA,
        pltpu.SemaphoreType.DMA,
        pltpu.SemaphoreType.DMA,
        pltpu.SemaphoreType.DMA,
        pltpu.SemaphoreType.DMA,
        pltpu.SemaphoreType.DMA,
    ),
)(_sc_edges_body)


@jax.jit
def kernel(x, edge_index, W_weight, W_bias, a_weight, a_bias):
    eflat = edge_index.astype(jnp.int32).reshape(2 * E)
    at = a_weight.reshape(2, D).astype(jnp.float32)
    ab = jnp.stack([a_bias[0], jnp.zeros((), jnp.float32)]).reshape(1, 2)
    b = W_bias.reshape(1, D)

    h, st = pl.pallas_call(
        _tc_prep_body,
        grid=(N // BN,),
        in_specs=[
            pl.BlockSpec((BN, D), lambda i: (i, 0)),
            pl.BlockSpec((D, D), lambda i: (0, 0)),
            pl.BlockSpec((1, D), lambda i: (0, 0)),
            pl.BlockSpec((2, D), lambda i: (0, 0)),
            pl.BlockSpec((1, 2), lambda i: (0, 0)),
        ],
        out_specs=[
            pl.BlockSpec((BN, D), lambda i: (i, 0)),
            pl.BlockSpec((BN, 2), lambda i: (i, 0)),
        ],
        out_shape=[
            jax.ShapeDtypeStruct((N, D), jnp.float32),
            jax.ShapeDtypeStruct((N, 2), jnp.float32),
        ],
    )(x, W_weight, b, at, ab)

    sti = lax.bitcast_convert_type(st.astype(jnp.bfloat16), jnp.int32)
    part, zp = _sc_edges(h, sti, eflat)

    out = pl.pallas_call(
        _tc_combine_body,
        grid=(N // BN,),
        in_specs=[
            pl.BlockSpec((1, BN, D), lambda i: (0, i, 0)),
            pl.BlockSpec((1, BN, D), lambda i: (1, i, 0)),
            pl.BlockSpec((4, 128), lambda i: (0, 0)),
        ],
        out_specs=pl.BlockSpec((BN, D), lambda i: (i, 0)),
        out_shape=jax.ShapeDtypeStruct((N, D), jnp.float32),
    )(part, part, zp.reshape(4, 128))
    return out
